# Initial kernel scaffold; baseline (speedup 1.0000x reference)
import jax
import jax.numpy as jnp
from jax.experimental import pallas as pl


def kernel(x, edge_index, W1_l, b1_l, W1_r, W2_l, b2_l, W2_r):
    raise NotImplementedError("write your pallas kernel here")


# speedup vs baseline: 5.0678x; 5.0678x over previous
import functools

import jax
import jax.numpy as jnp
from jax import lax
from jax.experimental import pallas as pl
from jax.experimental.pallas import tpu as pltpu
from jax.experimental.pallas import tpu_sc as plsc
from jax._src import core as _jax_core
from jax._src.lib.mlir import ir as _ir
from jax._src.lib.mlir.dialects import arith as _arith
from jax._src.pallas.mosaic import sc_lowering as _sc_lowering
from jax._src.pallas.mosaic import sc_primitives as _sc_primitives
from jax.experimental.mosaic.dialects import tpu as _tpu_dialect

_barrier2_p = _jax_core.Primitive("subcore_barrier2")
_barrier2_p.multiple_results = True


@_barrier2_p.def_effectful_abstract_eval
def _barrier2_abstract_eval():
    return (), {_sc_primitives._memory_effect}


@_sc_lowering.register_lowering_rule(_barrier2_p)
def _barrier2_lowering_rule(ctx):
    ix = _ir.IndexType.get()
    _tpu_dialect.barrier(_arith.constant(ix, _ir.IntegerAttr.get(ix, 1)))
    return ()


def _subcore_barrier2():
    _barrier2_p.bind()


_N = 10000
_E = 320000
_D = 128

_NC = 2
_NS = 16
_NW = _NC * _NS
_B = 80
_EPW = _E // _NW
_NCH = _EPW // _B
_NP = 10240
_RPS = _NP // _NS
_ZR = 32


def _sc_agg_body(with_counts, *refs):
    if with_counts:
        (feat, srci, dsti, out, cnt_out,
         sidx, didx, rows, zbuf, acc, sem,
         ones, zcbuf, cacc) = refs
    else:
        (feat, srci, dsti, out,
         sidx, didx, rows, zbuf, acc, sem) = refs

    cid = lax.axis_index("c")
    sid = lax.axis_index("s")
    wid = sid * _NC + cid
    row0 = sid * _RPS

    zv = jnp.zeros((16,), jnp.float32)
    for r in range(_ZR):
        for c in range(_D // 16):
            zbuf[r, pl.ds(c * 16, 16)] = zv
    if with_counts:
        ov = jnp.ones((16,), jnp.float32)
        for r in range(_B):
            ones[r, :] = ov
        for r in range(_ZR):
            zcbuf[r, :] = zv

    def zloop(i, carry):
        pltpu.sync_copy(zbuf, acc.at[pl.ds(row0 + i * _ZR, _ZR)])
        if with_counts:
            pltpu.sync_copy(zcbuf, cacc.at[pl.ds(row0 + i * _ZR, _ZR)])
        return carry

    lax.fori_loop(0, _RPS // _ZR, zloop, 0)
    plsc.subcore_barrier()

    def chunk(j, carry):
        base = (wid * _NCH + j) * _B
        pltpu.sync_copy(srci.at[pl.ds(base, _B)], sidx)
        pltpu.sync_copy(dsti.at[pl.ds(base, _B)], didx)
        pltpu.async_copy(feat.at[sidx], rows, sem).wait()
        pltpu.sync_copy(rows, acc.at[didx], add=True)
        if with_counts:
            pltpu.sync_copy(ones, cacc.at[didx], add=True)
        return carry

    lax.fori_loop(0, _NCH, chunk, 0)

    _subcore_barrier2()
    pltpu.sync_copy(acc.at[pl.ds(row0, _RPS)],
                    out.at[cid, pl.ds(row0, _RPS)])
    if with_counts:
        pltpu.sync_copy(cacc.at[pl.ds(row0, _RPS)],
                        cnt_out.at[cid, pl.ds(row0, _RPS)])


def _make_sc_agg(with_counts):
    outs = [jax.ShapeDtypeStruct((_NC, _NP, _D), jnp.float32)]
    scratch = [
        pltpu.VMEM((_B,), jnp.int32),
        pltpu.VMEM((_B,), jnp.int32),
        pltpu.VMEM((_B, _D), jnp.float32),
        pltpu.VMEM((_ZR, _D), jnp.float32),
        pltpu.VMEM_SHARED((_NP, _D), jnp.float32),
        pltpu.SemaphoreType.DMA,
    ]
    if with_counts:
        outs.append(jax.ShapeDtypeStruct((_NC, _NP, 16), jnp.float32))
        scratch += [
            pltpu.VMEM((_B, 16), jnp.float32),
            pltpu.VMEM((_ZR, 16), jnp.float32),
            pltpu.VMEM_SHARED((_NP, 16), jnp.float32),
        ]
    mesh = plsc.VectorSubcoreMesh(core_axis_name="c", subcore_axis_name="s")
    return pl.kernel(
        functools.partial(_sc_agg_body, with_counts),
        out_type=tuple(outs) if with_counts else outs[0],
        mesh=mesh,
        scratch_types=scratch,
        compiler_params=pltpu.CompilerParams(use_tc_tiling_on_sc=False),
    )


_sc_agg_cnt = _make_sc_agg(True)
_sc_agg = _make_sc_agg(False)

_RB = 1000


def _dense1_body(p_ref, c_ref, x_ref, wl_ref, wr_ref, b_ref, o_ref):
    s = p_ref[0] + p_ref[1]
    cnt = c_ref[0][:, 0:1] + c_ref[1][:, 0:1]
    mean = s / jnp.maximum(cnt, 1.0)
    h = (jnp.dot(mean, wl_ref[...], preferred_element_type=jnp.float32)
         + b_ref[...]
         + jnp.dot(x_ref[...], wr_ref[...], preferred_element_type=jnp.float32))
    o_ref[...] = jnp.maximum(h, 0.0)


def _dense2_body(p_ref, c_ref, h_ref, wl_ref, wr_ref, b_ref, o_ref):
    s = p_ref[0] + p_ref[1]
    cnt = c_ref[0][:, 0:1] + c_ref[1][:, 0:1]
    mean = s / jnp.maximum(cnt, 1.0)
    z = (jnp.dot(mean, wl_ref[...], preferred_element_type=jnp.float32)
         + b_ref[...]
         + jnp.dot(h_ref[...], wr_ref[...], preferred_element_type=jnp.float32))
    z = z - jnp.max(z, axis=1, keepdims=True)
    e = jnp.exp(z)
    o_ref[...] = e / jnp.sum(e, axis=1, keepdims=True)


def _make_dense(body):
    return pl.pallas_call(
        body,
        grid=(_N // _RB,),
        in_specs=[
            pl.BlockSpec((_NC, _RB, _D), lambda i: (0, i, 0)),
            pl.BlockSpec((_NC, _RB, 16), lambda i: (0, i, 0)),
            pl.BlockSpec((_RB, _D), lambda i: (i, 0)),
            pl.BlockSpec((_D, _D), lambda i: (0, 0)),
            pl.BlockSpec((_D, _D), lambda i: (0, 0)),
            pl.BlockSpec((1, _D), lambda i: (0, 0)),
        ],
        out_specs=pl.BlockSpec((_RB, _D), lambda i: (i, 0)),
        out_shape=jax.ShapeDtypeStruct((_N, _D), jnp.float32),
    )


_dense1 = _make_dense(_dense1_body)
_dense2 = _make_dense(_dense2_body)


def kernel(x, edge_index, W1_l, b1_l, W1_r, W2_l, b2_l, W2_r):
    ei = edge_index.astype(jnp.int32)
    srci = ei[0]
    dsti = ei[1]

    p1, cnt = _sc_agg_cnt(x, srci, dsti)
    h = _dense1(p1, cnt, x, W1_l.T, W1_r.T, b1_l.reshape(1, _D))
    p2 = _sc_agg(h, srci, dsti)
    return _dense2(p2, cnt, h, W2_l.T, W2_r.T, b2_l.reshape(1, _D))


# speedup vs baseline: 5.7569x; 1.1360x over previous
import functools

import jax
import jax.numpy as jnp
from jax import lax
from jax.experimental import pallas as pl
from jax.experimental.pallas import tpu as pltpu
from jax.experimental.pallas import tpu_sc as plsc
from jax._src import core as _jax_core
from jax._src.lib.mlir import ir as _ir
from jax._src.lib.mlir.dialects import arith as _arith
from jax._src.pallas.mosaic import sc_lowering as _sc_lowering
from jax._src.pallas.mosaic import sc_primitives as _sc_primitives
from jax.experimental.mosaic.dialects import tpu as _tpu_dialect

_barrier2_p = _jax_core.Primitive("subcore_barrier2")
_barrier2_p.multiple_results = True


@_barrier2_p.def_effectful_abstract_eval
def _barrier2_abstract_eval():
    return (), {_sc_primitives._memory_effect}


@_sc_lowering.register_lowering_rule(_barrier2_p)
def _barrier2_lowering_rule(ctx):
    ix = _ir.IndexType.get()
    _tpu_dialect.barrier(_arith.constant(ix, _ir.IntegerAttr.get(ix, 1)))
    return ()


def _subcore_barrier2():
    _barrier2_p.bind()


_N = 10000
_E = 320000
_D = 128

_NC = 2
_NS = 16
_NW = _NC * _NS
_B = 80
_EPW = _E // _NW
_NCH = _EPW // _B
_NP = 10240
_RPS = _NP // _NS
_ZR = 32


def _sc_agg_body(with_counts, *refs):
    if with_counts:
        (feat, srci, dsti, out, cnt_out,
         sidx, didx, rows, zbuf, acc, sem,
         sidx2, didx2, rows2, sem2,
         ones, zcbuf, cacc) = refs
    else:
        (feat, srci, dsti, out,
         sidx, didx, rows, zbuf, acc, sem,
         sidx2, didx2, rows2, sem2) = refs

    cid = lax.axis_index("c")
    sid = lax.axis_index("s")
    wid = sid * _NC + cid
    row0 = sid * _RPS

    zv = jnp.zeros((16,), jnp.float32)
    for r in range(_ZR):
        for c in range(_D // 16):
            zbuf[r, pl.ds(c * 16, 16)] = zv
    if with_counts:
        ov = jnp.ones((16,), jnp.float32)
        for r in range(_B):
            ones[r, :] = ov
        for r in range(_ZR):
            zcbuf[r, :] = zv

    def zloop(i, carry):
        pltpu.sync_copy(zbuf, acc.at[pl.ds(row0 + i * _ZR, _ZR)])
        if with_counts:
            pltpu.sync_copy(zcbuf, cacc.at[pl.ds(row0 + i * _ZR, _ZR)])
        return carry

    lax.fori_loop(0, _RPS // _ZR, zloop, 0)
    plsc.subcore_barrier()

    def chunk_pair(i, carry):
        base0 = (wid * _NCH + i * 2) * _B
        base1 = base0 + _B
        pltpu.sync_copy(srci.at[pl.ds(base0, _B)], sidx)
        pltpu.sync_copy(dsti.at[pl.ds(base0, _B)], didx)
        pltpu.sync_copy(srci.at[pl.ds(base1, _B)], sidx2)
        pltpu.sync_copy(dsti.at[pl.ds(base1, _B)], didx2)
        cp0 = pltpu.async_copy(feat.at[sidx], rows, sem)
        cp1 = pltpu.async_copy(feat.at[sidx2], rows2, sem2)
        cp0.wait()
        pltpu.sync_copy(rows, acc.at[didx], add=True)
        if with_counts:
            pltpu.sync_copy(ones, cacc.at[didx], add=True)
        cp1.wait()
        pltpu.sync_copy(rows2, acc.at[didx2], add=True)
        if with_counts:
            pltpu.sync_copy(ones, cacc.at[didx2], add=True)
        return carry

    lax.fori_loop(0, _NCH // 2, chunk_pair, 0)
    base = (wid * _NCH + _NCH - 1) * _B
    pltpu.sync_copy(srci.at[pl.ds(base, _B)], sidx)
    pltpu.sync_copy(dsti.at[pl.ds(base, _B)], didx)
    pltpu.async_copy(feat.at[sidx], rows, sem).wait()
    pltpu.sync_copy(rows, acc.at[didx], add=True)
    if with_counts:
        pltpu.sync_copy(ones, cacc.at[didx], add=True)

    _subcore_barrier2()
    pltpu.sync_copy(acc.at[pl.ds(row0, _RPS)],
                    out.at[cid, pl.ds(row0, _RPS)])
    if with_counts:
        pltpu.sync_copy(cacc.at[pl.ds(row0, _RPS)],
                        cnt_out.at[cid, pl.ds(row0, _RPS)])


def _make_sc_agg(with_counts):
    outs = [jax.ShapeDtypeStruct((_NC, _NP, _D), jnp.float32)]
    scratch = [
        pltpu.VMEM((_B,), jnp.int32),
        pltpu.VMEM((_B,), jnp.int32),
        pltpu.VMEM((_B, _D), jnp.float32),
        pltpu.VMEM((_ZR, _D), jnp.float32),
        pltpu.VMEM_SHARED((_NP, _D), jnp.float32),
        pltpu.SemaphoreType.DMA,
        pltpu.VMEM((_B,), jnp.int32),
        pltpu.VMEM((_B,), jnp.int32),
        pltpu.VMEM((_B, _D), jnp.float32),
        pltpu.SemaphoreType.DMA,
    ]
    if with_counts:
        outs.append(jax.ShapeDtypeStruct((_NC, _NP, 16), jnp.float32))
        scratch += [
            pltpu.VMEM((_B, 16), jnp.float32),
            pltpu.VMEM((_ZR, 16), jnp.float32),
            pltpu.VMEM_SHARED((_NP, 16), jnp.float32),
        ]
    mesh = plsc.VectorSubcoreMesh(core_axis_name="c", subcore_axis_name="s")
    return pl.kernel(
        functools.partial(_sc_agg_body, with_counts),
        out_type=tuple(outs) if with_counts else outs[0],
        mesh=mesh,
        scratch_types=scratch,
        compiler_params=pltpu.CompilerParams(use_tc_tiling_on_sc=False),
    )


_sc_agg_cnt = _make_sc_agg(True)
_sc_agg = _make_sc_agg(False)

_RB = 1000


def _dense1_body(p_ref, c_ref, x_ref, wl_ref, wr_ref, b_ref, o_ref):
    s = p_ref[0] + p_ref[1]
    cnt = c_ref[0][:, 0:1] + c_ref[1][:, 0:1]
    mean = s / jnp.maximum(cnt, 1.0)
    h = (jnp.dot(mean, wl_ref[...], preferred_element_type=jnp.float32)
         + b_ref[...]
         + jnp.dot(x_ref[...], wr_ref[...], preferred_element_type=jnp.float32))
    o_ref[...] = jnp.maximum(h, 0.0)


def _dense2_body(p_ref, c_ref, h_ref, wl_ref, wr_ref, b_ref, o_ref):
    s = p_ref[0] + p_ref[1]
    cnt = c_ref[0][:, 0:1] + c_ref[1][:, 0:1]
    mean = s / jnp.maximum(cnt, 1.0)
    z = (jnp.dot(mean, wl_ref[...], preferred_element_type=jnp.float32)
         + b_ref[...]
         + jnp.dot(h_ref[...], wr_ref[...], preferred_element_type=jnp.float32))
    z = z - jnp.max(z, axis=1, keepdims=True)
    e = jnp.exp(z)
    o_ref[...] = e / jnp.sum(e, axis=1, keepdims=True)


def _make_dense(body):
    return pl.pallas_call(
        body,
        grid=(_N // _RB,),
        in_specs=[
            pl.BlockSpec((_NC, _RB, _D), lambda i: (0, i, 0)),
            pl.BlockSpec((_NC, _RB, 16), lambda i: (0, i, 0)),
            pl.BlockSpec((_RB, _D), lambda i: (i, 0)),
            pl.BlockSpec((_D, _D), lambda i: (0, 0)),
            pl.BlockSpec((_D, _D), lambda i: (0, 0)),
            pl.BlockSpec((1, _D), lambda i: (0, 0)),
        ],
        out_specs=pl.BlockSpec((_RB, _D), lambda i: (i, 0)),
        out_shape=jax.ShapeDtypeStruct((_N, _D), jnp.float32),
    )


_dense1 = _make_dense(_dense1_body)
_dense2 = _make_dense(_dense2_body)


def kernel(x, edge_index, W1_l, b1_l, W1_r, W2_l, b2_l, W2_r):
    ei = edge_index.astype(jnp.int32)
    srci = ei[0]
    dsti = ei[1]

    p1, cnt = _sc_agg_cnt(x, srci, dsti)
    h = _dense1(p1, cnt, x, W1_l.T, W1_r.T, b1_l.reshape(1, _D))
    p2 = _sc_agg(h, srci, dsti)
    return _dense2(p2, cnt, h, W2_l.T, W2_r.T, b2_l.reshape(1, _D))
